# baseline (speedup 1.0000x reference)
import dataclasses

import jax
import jax.numpy as jnp
from jax import lax
from jax.experimental import pallas as pl
from jax.experimental.pallas import tpu as pltpu
from jax.experimental.pallas import tpu_sc as plsc

_PROJ = 64
_DEPTH = 20
_L = 16
_PAD = 32


def _sc_body(x_idx_hbm, dir_hbm, label_hbm, emb1_hbm, emb2_hbm,
             out_hbm, tgt_hbm,
             xv, dv, lv, proj_v, rows_v, outv, tgtv,
             sem0, sem1, sem2):
    cid = lax.axis_index("c")
    sid = lax.axis_index("s")

    @pl.when(jnp.logical_and(cid == 0, sid == 0))
    def _():
        cp_x = pltpu.async_copy(x_idx_hbm, xv, sem0)
        cp_d = pltpu.async_copy(dir_hbm, dv, sem1)
        cp_l = pltpu.async_copy(label_hbm.at[0], lv.at[pl.ds(0, _DEPTH)], sem2)
        cp_x.wait()
        cp_p = pltpu.async_copy(emb1_hbm.at[xv], proj_v, sem0)
        cp_d.wait()
        cp_r = pltpu.async_copy(emb2_hbm.at[dv], rows_v, sem1)
        cp_p.wait()
        cp_r.wait()
        cp_l.wait()

        def bf16_trunc(v):
            b = plsc.bitcast(v, jnp.uint32)
            r = b + jnp.uint32(0x7FFF) + ((b >> jnp.uint32(16)) & jnp.uint32(1))
            return plsc.bitcast(r & jnp.uint32(0xFFFF0000), jnp.float32)

        proj = [bf16_trunc(proj_v[0, pl.ds(k * _L, _L)])
                for k in range(_PROJ // _L)]
        lanes = lax.iota(jnp.int32, _L)
        acc = [jnp.zeros((_L,), jnp.float32) for _ in range(_PAD // _L)]
        for i in range(_DEPTH):
            d = proj[0] * bf16_trunc(rows_v[i, pl.ds(0, _L)])
            for k in range(1, _PROJ // _L):
                d = d + proj[k] * bf16_trunc(rows_v[i, pl.ds(k * _L, _L)])
            s = jnp.sum(d)
            acc[i // _L] = jnp.where(lanes == (i % _L), s, acc[i // _L])

        for c in range(_PAD // _L):
            out = 1.0 / (1.0 + jnp.exp(-acc[c]))
            mask = jnp.where(out >= 0.5, 1, 0)
            lab = lv[pl.ds(c * _L, _L)]
            tgt = jnp.where(mask == lab, 1, 0)
            outv[pl.ds(c * _L, _L)] = out
            tgtv[pl.ds(c * _L, _L)] = tgt

        cp_o = pltpu.async_copy(outv.at[pl.ds(0, _DEPTH)], out_hbm.at[0], sem0)
        cp_t = pltpu.async_copy(tgtv.at[pl.ds(0, _DEPTH)], tgt_hbm.at[0], sem1)
        cp_o.wait()
        cp_t.wait()


def _compiler_params():
    return pltpu.CompilerParams(needs_layout_passes=False,
                                use_tc_tiling_on_sc=False)


def _run(x_idx, dir_path, label, emb1, emb2):
    call = pl.kernel(
        _sc_body,
        compiler_params=_compiler_params(),
        out_type=(jax.ShapeDtypeStruct((1, _DEPTH), jnp.float32),
                  jax.ShapeDtypeStruct((1, _DEPTH), jnp.int32)),
        mesh=plsc.VectorSubcoreMesh(core_axis_name="c", subcore_axis_name="s",
                                    num_cores=2, num_subcores=16),
        scratch_types=[
            pltpu.VMEM((1,), jnp.int32),
            pltpu.VMEM((_DEPTH,), jnp.int32),
            pltpu.VMEM((_PAD,), jnp.int32),
            pltpu.VMEM((1, _PROJ), jnp.float32),
            pltpu.VMEM((_DEPTH, _PROJ), jnp.float32),
            pltpu.VMEM((_PAD,), jnp.float32),
            pltpu.VMEM((_PAD,), jnp.int32),
            pltpu.SemaphoreType.DMA,
            pltpu.SemaphoreType.DMA,
            pltpu.SemaphoreType.DMA,
        ],
    )
    return call(x_idx, dir_path, label, emb1, emb2)


def kernel(x_idx, dir_path, label, emb1, emb2):
    out, tgt = _run(x_idx.astype(jnp.int32), dir_path.astype(jnp.int32),
                    label.astype(jnp.int32), emb1, emb2)
    return (out, tgt.astype(label.dtype))


# speedup vs baseline: 1.5894x; 1.5894x over previous
import dataclasses

import jax
import jax.numpy as jnp
from jax import lax
from jax.experimental import pallas as pl
from jax.experimental.pallas import tpu as pltpu
from jax.experimental.pallas import tpu_sc as plsc

_PROJ = 64
_DEPTH = 20
_L = 16
_PAD = 32


def _sc_body(x_idx_hbm, dir_hbm, label_hbm, emb1_hbm, emb2_hbm,
             out_hbm, tgt_hbm,
             xs, dsm, lv, proj_v, rows_v, outv, tgtv,
             sem0, sem1, sem2):
    cid = lax.axis_index("c")
    sid = lax.axis_index("s")

    @pl.when(jnp.logical_and(cid == 0, sid == 0))
    def _():
        cp_x = pltpu.async_copy(x_idx_hbm, xs.at[pl.ds(0, 1)], sem0)
        cp_d = pltpu.async_copy(dir_hbm, dsm.at[pl.ds(0, _DEPTH)], sem1)
        cp_l = pltpu.async_copy(label_hbm.at[0], lv.at[pl.ds(0, _DEPTH)], sem2)
        cp_x.wait()
        x0 = xs[pl.ds(0, _L)][0]
        cp_p = pltpu.async_copy(emb1_hbm.at[pl.ds(x0, 1)], proj_v, sem0)
        cp_d.wait()
        d_lanes = [dsm[pl.ds(0, _L)], dsm[pl.ds(_L, _L)]]
        row_cps = [
            pltpu.async_copy(emb2_hbm.at[pl.ds(d_lanes[i // _L][i % _L], 1)],
                             rows_v.at[pl.ds(i, 1)], sem1)
            for i in range(_DEPTH)
        ]
        cp_p.wait()
        for cp in row_cps:
            cp.wait()
        cp_l.wait()

        def bf16_trunc(v):
            b = plsc.bitcast(v, jnp.uint32)
            r = b + jnp.uint32(0x7FFF) + ((b >> jnp.uint32(16)) & jnp.uint32(1))
            return plsc.bitcast(r & jnp.uint32(0xFFFF0000), jnp.float32)

        proj = [bf16_trunc(proj_v[0, pl.ds(k * _L, _L)])
                for k in range(_PROJ // _L)]
        lanes = lax.iota(jnp.int32, _L)
        acc = [jnp.zeros((_L,), jnp.float32) for _ in range(_PAD // _L)]
        for i in range(_DEPTH):
            d = proj[0] * bf16_trunc(rows_v[i, pl.ds(0, _L)])
            for k in range(1, _PROJ // _L):
                d = d + proj[k] * bf16_trunc(rows_v[i, pl.ds(k * _L, _L)])
            s = jnp.sum(d)
            acc[i // _L] = jnp.where(lanes == (i % _L), s, acc[i // _L])

        for c in range(_PAD // _L):
            out = 1.0 / (1.0 + jnp.exp(-acc[c]))
            mask = jnp.where(out >= 0.5, 1, 0)
            lab = lv[pl.ds(c * _L, _L)]
            tgt = jnp.where(mask == lab, 1, 0)
            outv[pl.ds(c * _L, _L)] = out
            tgtv[pl.ds(c * _L, _L)] = tgt

        cp_o = pltpu.async_copy(outv.at[pl.ds(0, _DEPTH)], out_hbm.at[0], sem0)
        cp_t = pltpu.async_copy(tgtv.at[pl.ds(0, _DEPTH)], tgt_hbm.at[0], sem1)
        cp_o.wait()
        cp_t.wait()


def _compiler_params():
    return pltpu.CompilerParams(needs_layout_passes=False)


def _run(x_idx, dir_path, label, emb1, emb2):
    call = pl.kernel(
        _sc_body,
        compiler_params=_compiler_params(),
        out_type=(jax.ShapeDtypeStruct((1, _DEPTH), jnp.float32),
                  jax.ShapeDtypeStruct((1, _DEPTH), jnp.int32)),
        mesh=plsc.VectorSubcoreMesh(core_axis_name="c", subcore_axis_name="s",
                                    num_cores=2, num_subcores=16),
        scratch_types=[
            pltpu.VMEM((_L,), jnp.int32),
            pltpu.VMEM((_PAD,), jnp.int32),
            pltpu.VMEM((_PAD,), jnp.int32),
            pltpu.VMEM((1, _PROJ), jnp.float32),
            pltpu.VMEM((_DEPTH, _PROJ), jnp.float32),
            pltpu.VMEM((_PAD,), jnp.float32),
            pltpu.VMEM((_PAD,), jnp.int32),
            pltpu.SemaphoreType.DMA,
            pltpu.SemaphoreType.DMA,
            pltpu.SemaphoreType.DMA,
        ],
    )
    return call(x_idx, dir_path, label, emb1, emb2)


def kernel(x_idx, dir_path, label, emb1, emb2):
    out, tgt = _run(x_idx.astype(jnp.int32), dir_path.astype(jnp.int32),
                    label.astype(jnp.int32), emb1, emb2)
    return (out, tgt.astype(label.dtype))


# speedup vs baseline: 36.6472x; 23.0575x over previous
import jax
import jax.numpy as jnp
from jax import lax
from jax.experimental import pallas as pl
from jax.experimental.pallas import tpu as pltpu
from jax.experimental.pallas import tpu_sc as plsc

_PROJ = 64
_DEPTH = 20
_L = 16
_PAD = 32
_TILE = 128
_WAVE = 12


def _bf16_trunc(v):
    b = plsc.bitcast(v, jnp.uint32)
    r = b + jnp.uint32(0x7FFF) + ((b >> jnp.uint32(16)) & jnp.uint32(1))
    return plsc.bitcast(r & jnp.uint32(0xFFFF0000), jnp.float32)


def _sc_body(x_idx_hbm, dir_hbm, label_hbm, e1t_hbm, e2t_hbm,
             out_hbm, tgt_hbm,
             xs, dsm, lv, blocks, outv, tgtv,
             sem0, sem1, sem2):
    cid = lax.axis_index("c")
    sid = lax.axis_index("s")

    @pl.when(jnp.logical_and(cid == 0, sid == 0))
    def _():
        cp_x = pltpu.async_copy(x_idx_hbm, xs.at[pl.ds(0, 1)], sem0)
        cp_d = pltpu.async_copy(dir_hbm, dsm.at[pl.ds(0, _DEPTH)], sem1)
        cp_l = pltpu.async_copy(label_hbm.at[0], lv.at[pl.ds(0, _DEPTH)], sem2)
        cp_x.wait()
        cp_d.wait()

        x0 = xs[pl.ds(0, _L)][0]
        d_off = []
        d_col = []
        for c in range(_PAD // _L):
            dvec = dsm[pl.ds(c * _L, _L)]
            ovec = (dvec >> jnp.int32(7)) << jnp.int32(7)
            cvec = dvec & jnp.int32(_TILE - 1)
            d_off += [ovec[j] for j in range(_L)]
            d_col += [cvec[j] for j in range(_L)]

        fetches = [(e1t_hbm, (x0 >> jnp.int32(7)) << jnp.int32(7),
                    x0 & jnp.int32(_TILE - 1))]
        fetches += [(e2t_hbm, d_off[i], d_col[i]) for i in range(_DEPTH)]

        rows = [lax.iota(jnp.int32, _L) + jnp.int32(k * _L)
                for k in range(_PROJ // _L)]
        lanes = lax.iota(jnp.int32, _L)

        def extract_column(slot, col):
            cvec = jnp.full((_L,), col, jnp.int32)
            svec = jnp.full((_L,), slot, jnp.int32)
            return [_bf16_trunc(plsc.load_gather(blocks, [svec, rows[k], cvec]))
                    for k in range(_PROJ // _L)]

        proj = None
        acc = [jnp.zeros((_L,), jnp.float32) for _ in range(_PAD // _L)]
        done = 0
        while done < len(fetches):
            wave = fetches[done:done + _WAVE]
            cps = [
                pltpu.async_copy(src.at[:, pl.ds(pl.multiple_of(off, _TILE),
                                                 _TILE)],
                                 blocks.at[slot], sem1)
                for slot, (src, off, _) in enumerate(wave)
            ]
            for cp in cps:
                cp.wait()
            for slot, (_, _, col) in enumerate(wave):
                chunks = extract_column(slot, col)
                if proj is None:
                    proj = chunks
                    continue
                i = done + slot - 1
                d = proj[0] * chunks[0]
                for k in range(1, _PROJ // _L):
                    d = d + proj[k] * chunks[k]
                s = jnp.sum(d)
                acc[i // _L] = jnp.where(lanes == (i % _L), s, acc[i // _L])
            done += len(wave)

        cp_l.wait()
        for c in range(_PAD // _L):
            out = 1.0 / (1.0 + jnp.exp(-acc[c]))
            mask = jnp.where(out >= 0.5, 1, 0)
            lab = lv[pl.ds(c * _L, _L)]
            tgt = jnp.where(mask == lab, 1, 0)
            outv[pl.ds(c * _L, _L)] = out
            tgtv[pl.ds(c * _L, _L)] = tgt

        cp_o = pltpu.async_copy(outv.at[pl.ds(0, _DEPTH)], out_hbm.at[0], sem0)
        cp_t = pltpu.async_copy(tgtv.at[pl.ds(0, _DEPTH)], tgt_hbm.at[0], sem2)
        cp_o.wait()
        cp_t.wait()


def _compiler_params():
    return pltpu.CompilerParams(needs_layout_passes=False)


def _run(x_idx, dir_path, label, emb1_t, emb2_t):
    call = pl.kernel(
        _sc_body,
        compiler_params=_compiler_params(),
        out_type=(jax.ShapeDtypeStruct((1, _DEPTH), jnp.float32),
                  jax.ShapeDtypeStruct((1, _DEPTH), jnp.int32)),
        mesh=plsc.VectorSubcoreMesh(core_axis_name="c", subcore_axis_name="s",
                                    num_cores=2, num_subcores=16),
        scratch_types=[
            pltpu.VMEM((_L,), jnp.int32),
            pltpu.VMEM((_PAD,), jnp.int32),
            pltpu.VMEM((_PAD,), jnp.int32),
            pltpu.VMEM((_WAVE, _PROJ, _TILE), jnp.float32),
            pltpu.VMEM((_PAD,), jnp.float32),
            pltpu.VMEM((_PAD,), jnp.int32),
            pltpu.SemaphoreType.DMA,
            pltpu.SemaphoreType.DMA,
            pltpu.SemaphoreType.DMA,
        ],
    )
    return call(x_idx, dir_path, label, emb1_t, emb2_t)


def kernel(x_idx, dir_path, label, emb1, emb2):
    out, tgt = _run(x_idx.astype(jnp.int32), dir_path.astype(jnp.int32),
                    label.astype(jnp.int32), emb1.T, emb2.T)
    return (out, tgt.astype(label.dtype))


# speedup vs baseline: 42.0978x; 1.1487x over previous
import jax
import jax.numpy as jnp
from jax import lax
from jax.experimental import pallas as pl
from jax.experimental.pallas import tpu as pltpu
from jax.experimental.pallas import tpu_sc as plsc

_PROJ = 64
_DEPTH = 20
_L = 16
_PAD = 32
_TILE = 128
_SPLIT = 8
_MAXFETCH = 1 + (_DEPTH - _SPLIT)


def _bf16_trunc(v):
    b = plsc.bitcast(v, jnp.uint32)
    r = b + jnp.uint32(0x7FFF) + ((b >> jnp.uint32(16)) & jnp.uint32(1))
    return plsc.bitcast(r & jnp.uint32(0xFFFF0000), jnp.float32)


def _sc_body(x_idx_hbm, dir_hbm, label_hbm, e1t_hbm, e2t_hbm,
             out_hbm, tgt_hbm,
             xs, dsm, lv, blocks, outv, tgtv,
             sem0, sem1, sem2):
    cid = lax.axis_index("c")
    sid = lax.axis_index("s")

    def half(lo, hi):
        cp_x = pltpu.async_copy(x_idx_hbm, xs.at[pl.ds(0, 1)], sem0)
        cp_d = pltpu.async_copy(dir_hbm, dsm.at[pl.ds(0, _DEPTH)], sem1)
        cp_l = pltpu.async_copy(label_hbm.at[0], lv.at[pl.ds(0, _DEPTH)], sem2)
        cp_x.wait()
        cp_d.wait()

        x0 = xs[pl.ds(0, _L)][0]
        d_off, d_col = {}, {}
        for c in range(_PAD // _L):
            if not any(lo <= i < hi for i in range(c * _L, (c + 1) * _L)):
                continue
            dvec = dsm[pl.ds(c * _L, _L)]
            ovec = (dvec >> jnp.int32(7)) << jnp.int32(7)
            cvec = dvec & jnp.int32(_TILE - 1)
            for j in range(_L):
                i = c * _L + j
                if lo <= i < hi:
                    d_off[i] = ovec[j]
                    d_col[i] = cvec[j]

        fetches = [(e1t_hbm, (x0 >> jnp.int32(7)) << jnp.int32(7),
                    x0 & jnp.int32(_TILE - 1))]
        fetches += [(e2t_hbm, d_off[i], d_col[i]) for i in range(lo, hi)]

        cps = [
            pltpu.async_copy(src.at[:, pl.ds(pl.multiple_of(off, _TILE),
                                             _TILE)],
                             blocks.at[slot], sem1)
            for slot, (src, off, _) in enumerate(fetches)
        ]
        for cp in cps:
            cp.wait()

        rows = [lax.iota(jnp.int32, _L) + jnp.int32(k * _L)
                for k in range(_PROJ // _L)]
        lanes = lax.iota(jnp.int32, _L)

        def extract_column(slot, col):
            cvec = jnp.full((_L,), col, jnp.int32)
            svec = jnp.full((_L,), slot, jnp.int32)
            return [_bf16_trunc(plsc.load_gather(blocks, [svec, rows[k], cvec]))
                    for k in range(_PROJ // _L)]

        proj = extract_column(0, fetches[0][2])
        acc = [jnp.zeros((_L,), jnp.float32) for _ in range(_PAD // _L)]
        for slot, i in enumerate(range(lo, hi), start=1):
            chunks = extract_column(slot, fetches[slot][2])
            d = proj[0] * chunks[0]
            for k in range(1, _PROJ // _L):
                d = d + proj[k] * chunks[k]
            s = jnp.sum(d)
            acc[i // _L] = jnp.where(lanes == (i % _L), s, acc[i // _L])

        cp_l.wait()
        for c in range(_PAD // _L):
            if not any(lo <= i < hi for i in range(c * _L, (c + 1) * _L)):
                continue
            out = 1.0 / (1.0 + jnp.exp(-acc[c]))
            mask = jnp.where(out >= 0.5, 1, 0)
            lab = lv[pl.ds(c * _L, _L)]
            tgt = jnp.where(mask == lab, 1, 0)
            outv[pl.ds(c * _L, _L)] = out
            tgtv[pl.ds(c * _L, _L)] = tgt

        n = hi - lo
        cp_o = pltpu.async_copy(outv.at[pl.ds(lo, n)],
                                out_hbm.at[0].at[pl.ds(lo, n)], sem0)
        cp_t = pltpu.async_copy(tgtv.at[pl.ds(lo, n)],
                                tgt_hbm.at[0].at[pl.ds(lo, n)], sem2)
        cp_o.wait()
        cp_t.wait()

    @pl.when(jnp.logical_and(cid == 0, sid == 0))
    def _():
        half(0, _SPLIT)

    @pl.when(jnp.logical_and(cid == 1, sid == 0))
    def _():
        half(_SPLIT, _DEPTH)


def _compiler_params():
    return pltpu.CompilerParams(needs_layout_passes=False)


def _run(x_idx, dir_path, label, emb1_t, emb2_t):
    call = pl.kernel(
        _sc_body,
        compiler_params=_compiler_params(),
        out_type=(jax.ShapeDtypeStruct((1, _DEPTH), jnp.float32),
                  jax.ShapeDtypeStruct((1, _DEPTH), jnp.int32)),
        mesh=plsc.VectorSubcoreMesh(core_axis_name="c", subcore_axis_name="s",
                                    num_cores=2, num_subcores=16),
        scratch_types=[
            pltpu.VMEM((_L,), jnp.int32),
            pltpu.VMEM((_PAD,), jnp.int32),
            pltpu.VMEM((_PAD,), jnp.int32),
            pltpu.VMEM((_MAXFETCH, _PROJ, _TILE), jnp.float32),
            pltpu.VMEM((_PAD,), jnp.float32),
            pltpu.VMEM((_PAD,), jnp.int32),
            pltpu.SemaphoreType.DMA,
            pltpu.SemaphoreType.DMA,
            pltpu.SemaphoreType.DMA,
        ],
    )
    return call(x_idx, dir_path, label, emb1_t, emb2_t)


def kernel(x_idx, dir_path, label, emb1, emb2):
    out, tgt = _run(x_idx.astype(jnp.int32), dir_path.astype(jnp.int32),
                    label.astype(jnp.int32), emb1.T, emb2.T)
    return (out, tgt.astype(label.dtype))
